# Initial kernel scaffold; baseline (speedup 1.0000x reference)
import jax
import jax.numpy as jnp
from jax.experimental import pallas as pl


def kernel(x, w_ih_0, w_hh_0, b_0, w_ih_1, w_hh_1, b_1):
    raise NotImplementedError("write your pallas kernel here")


# speedup vs baseline: 4.4775x; 4.4775x over previous
import functools

import jax
import jax.numpy as jnp
from jax.experimental import pallas as pl
from jax.experimental.pallas import tpu as pltpu


def _premul_kernel(x_ref, w_ref, b_ref, o_ref):
    o_ref[...] = (
        jnp.dot(x_ref[...], w_ref[...], preferred_element_type=jnp.float32)
        + b_ref[...]
    )


def _premul(x2d, w_t, b):
    M, K = x2d.shape
    N = w_t.shape[1]
    bm, bn = min(1024, M), min(1024, N)
    return pl.pallas_call(
        _premul_kernel,
        out_shape=jax.ShapeDtypeStruct((M, N), jnp.float32),
        grid=(M // bm, N // bn),
        in_specs=[
            pl.BlockSpec((bm, K), lambda i, j: (i, 0)),
            pl.BlockSpec((K, bn), lambda i, j: (0, j)),
            pl.BlockSpec((1, bn), lambda i, j: (0, j)),
        ],
        out_specs=pl.BlockSpec((bm, bn), lambda i, j: (i, j)),
        compiler_params=pltpu.CompilerParams(
            dimension_semantics=("parallel", "parallel"),
        ),
        name="sublstm_premul",
    )(x2d, w_t, b.reshape(1, N))


def _rec_kernel(H, pre_ref, w_ref, o_ref, h_s, c_s):
    t = pl.program_id(0)

    @pl.when(t == 0)
    def _():
        h_s[...] = jnp.zeros_like(h_s)
        c_s[...] = jnp.zeros_like(c_s)

    gates = jax.nn.sigmoid(
        pre_ref[0]
        + jnp.dot(h_s[...], w_ref[...], preferred_element_type=jnp.float32)
    )
    i_g = gates[:, :H]
    o_g = gates[:, H : 2 * H]
    z_g = gates[:, 2 * H : 3 * H]
    f_g = gates[:, 3 * H :]
    c = c_s[...] * f_g + z_g - i_g
    h = jax.nn.sigmoid(c) - o_g
    c_s[...] = c
    h_s[...] = h
    o_ref[0] = h


def _recurrence(pre, w_hh_t):
    T, B, G = pre.shape
    H = w_hh_t.shape[0]
    return pl.pallas_call(
        functools.partial(_rec_kernel, H),
        out_shape=jax.ShapeDtypeStruct((T, B, H), jnp.float32),
        grid=(T,),
        in_specs=[
            pl.BlockSpec((1, B, G), lambda t: (t, 0, 0)),
            pl.BlockSpec((H, G), lambda t: (0, 0)),
        ],
        out_specs=pl.BlockSpec((1, B, H), lambda t: (t, 0, 0)),
        scratch_shapes=[
            pltpu.VMEM((B, H), jnp.float32),
            pltpu.VMEM((B, H), jnp.float32),
        ],
        compiler_params=pltpu.CompilerParams(
            dimension_semantics=("arbitrary",),
        ),
        name="sublstm_recurrence",
    )(pre, w_hh_t)


def kernel(x, w_ih_0, w_hh_0, b_0, w_ih_1, w_hh_1, b_1):
    T, B, I = x.shape
    H = w_hh_0.shape[1]
    pre1 = _premul(x.reshape(T * B, I), w_ih_0.T, b_0)
    h1 = _recurrence(pre1.reshape(T, B, 4 * H), w_hh_0.T)
    pre2 = _premul(h1.reshape(T * B, H), w_ih_1.T, b_1)
    h2 = _recurrence(pre2.reshape(T, B, 4 * H), w_hh_1.T)
    return h2
